# Initial kernel scaffold; baseline (speedup 1.0000x reference)
import jax
import jax.numpy as jnp
from jax.experimental import pallas as pl


def kernel(x, edge_index, edge_type, norm, weight):
    raise NotImplementedError("write your pallas kernel here")


# speedup vs baseline: 82.2438x; 82.2438x over previous
import functools

import jax
import jax.numpy as jnp
from jax import lax
from jax.experimental import pallas as pl
from jax.experimental.pallas import tpu as pltpu
from jax.experimental.pallas import tpu_sc as plsc

N = 10000
E = 320000
F = 128
R = 8
SUB = 16

NC = 2
NS = 16
NW = NC * NS
EPW = E // NW
K = 80
CH = EPW // K
NSEG = 5
SCH = CH // NSEG
NP = 10240
RPT = NP // NS
ZR = 64
BN = 2000


def _table_body(x_ref, w_ref, out_ref):
    r = pl.program_id(0)
    w2 = w_ref[r]
    bi = lax.broadcasted_iota(jnp.int32, (F, SUB), 0) // SUB
    cols = [jnp.where(bi == b, w2, 0.0) for b in range(F // SUB)]
    bd = jnp.concatenate(cols, axis=1)
    out_ref[...] = jnp.dot(x_ref[...], bd, preferred_element_type=jnp.float32)


def _make_table(x, weight):
    grid = (R, N // BN)
    return pl.pallas_call(
        _table_body,
        grid=grid,
        in_specs=[
            pl.BlockSpec((BN, F), lambda r, n: (n, 0)),
            pl.BlockSpec((R, F, SUB), lambda r, n: (0, 0, 0)),
        ],
        out_specs=pl.BlockSpec((BN, F), lambda r, n: (r * (N // BN) + n, 0)),
        out_shape=jax.ShapeDtypeStruct((R * N, F), jnp.float32),
    )(x, weight.reshape(R, F, SUB))


def _sc_body(table, srcT, etT, dstT, out,
             src_v, et_v, dst_v, idx_v, g_v, zbuf, acc, sem):
    c = lax.axis_index("c")
    s = lax.axis_index("s")
    wid = c * NS + s
    row0 = s * RPT

    zeros16 = jnp.zeros((16,), jnp.float32)

    @pl.loop(0, ZR)
    def _(r):
        for j in range(F // 16):
            zbuf[r, pl.ds(j * 16, 16)] = zeros16

    @pl.loop(0, RPT // ZR)
    def _(i):
        pltpu.sync_copy(zbuf, acc.at[pl.ds(row0 + i * ZR, ZR)])

    plsc.subcore_barrier()

    @pl.loop(0, NSEG)
    def _(seg):
        pltpu.sync_copy(srcT.at[wid, seg], src_v)
        pltpu.sync_copy(etT.at[wid, seg], et_v)
        pltpu.sync_copy(dstT.at[wid, seg], dst_v)

        @pl.loop(0, SCH)
        def _(g):
            for j in range(K // 16):
                sl = pl.ds(j * 16, 16)
                idx_v[sl] = et_v[g, sl] * N + src_v[g, sl]
            pltpu.async_copy(table.at[idx_v], g_v, sem).wait()
            pltpu.sync_copy(g_v, acc.at[dst_v.at[g]], add=True)

    plsc.subcore_barrier()

    @pl.loop(0, RPT // ZR)
    def _(i):
        r0 = row0 + i * ZR
        pltpu.sync_copy(acc.at[pl.ds(r0, ZR)], zbuf)
        pltpu.sync_copy(zbuf, out.at[c, pl.ds(r0, ZR)])


def _sc_scatter(table, srcT, etT, dstT):
    mesh = plsc.VectorSubcoreMesh(core_axis_name="c", subcore_axis_name="s")
    kern = pl.kernel(
        _sc_body,
        out_type=jax.ShapeDtypeStruct((NC, NP, F), jnp.float32),
        mesh=mesh,
        scratch_types=[
            pltpu.VMEM((SCH, K), jnp.int32),
            pltpu.VMEM((SCH, K), jnp.int32),
            pltpu.VMEM((SCH, K), jnp.int32),
            pltpu.VMEM((K,), jnp.int32),
            pltpu.VMEM((K, F), jnp.float32),
            pltpu.VMEM((ZR, F), jnp.float32),
            pltpu.VMEM_SHARED((NP, F), jnp.float32),
            pltpu.SemaphoreType.DMA,
        ],
    )
    return kern(table, srcT, etT, dstT)


def _combine_body(p_ref, n_ref, o_ref):
    o_ref[...] = (p_ref[0] + p_ref[1]) * n_ref[...]


def _combine(partial, norm):
    grid = (N // BN,)
    return pl.pallas_call(
        _combine_body,
        grid=grid,
        in_specs=[
            pl.BlockSpec((NC, BN, F), lambda n: (0, n, 0)),
            pl.BlockSpec((BN, 1), lambda n: (n, 0)),
        ],
        out_specs=pl.BlockSpec((BN, F), lambda n: (n, 0)),
        out_shape=jax.ShapeDtypeStruct((N, F), jnp.float32),
    )(partial, norm)


def kernel(x, edge_index, edge_type, norm, weight):
    table = _make_table(x, weight)
    srcT = edge_index[0].reshape(NW, NSEG, SCH, K)
    dstT = edge_index[1].reshape(NW, NSEG, SCH, K)
    etT = edge_type.reshape(NW, NSEG, SCH, K)
    partial = _sc_scatter(table, srcT, etT, dstT)
    return _combine(partial, norm)


# speedup vs baseline: 114.6061x; 1.3935x over previous
import functools

import jax
import jax.numpy as jnp
from jax import lax
from jax.experimental import pallas as pl
from jax.experimental.pallas import tpu as pltpu
from jax.experimental.pallas import tpu_sc as plsc

N = 10000
E = 320000
F = 128
R = 8
SUB = 16

NC = 2
NS = 16
NW = NC * NS
EPW = E // NW
K = 80
CH = EPW // K
NSEG = 5
SCH = CH // NSEG
NP = 10240
RPT = NP // NS
ZR = 64
BN = 2000


def _table_body(x_ref, w_ref, out_ref):
    r = pl.program_id(0)
    w2 = w_ref[r]
    bi = lax.broadcasted_iota(jnp.int32, (F, SUB), 0) // SUB
    cols = [jnp.where(bi == b, w2, 0.0) for b in range(F // SUB)]
    bd = jnp.concatenate(cols, axis=1)
    out_ref[...] = jnp.dot(x_ref[...], bd, preferred_element_type=jnp.float32)


def _make_table(x, weight):
    grid = (R, N // BN)
    return pl.pallas_call(
        _table_body,
        grid=grid,
        in_specs=[
            pl.BlockSpec((BN, F), lambda r, n: (n, 0)),
            pl.BlockSpec((R, F, SUB), lambda r, n: (0, 0, 0)),
        ],
        out_specs=pl.BlockSpec((BN, F), lambda r, n: (r * (N // BN) + n, 0)),
        out_shape=jax.ShapeDtypeStruct((R * N, F), jnp.float32),
    )(x, weight.reshape(R, F, SUB))


def _sc_body(table, srcT, etT, dstT, out,
             src_v, et_v, dst_v, idx_v, gA, gB, zbuf, acc, semA, semB):
    c = lax.axis_index("c")
    s = lax.axis_index("s")
    wid = c * NS + s
    row0 = s * RPT

    zeros16 = jnp.zeros((16,), jnp.float32)

    @pl.loop(0, ZR)
    def _(r):
        for j in range(F // 16):
            zbuf[r, pl.ds(j * 16, 16)] = zeros16

    @pl.loop(0, RPT // ZR)
    def _(i):
        pltpu.sync_copy(zbuf, acc.at[pl.ds(row0 + i * ZR, ZR)])

    plsc.subcore_barrier()

    @pl.loop(0, NSEG)
    def _(seg):
        pltpu.sync_copy(srcT.at[wid, seg], src_v)
        pltpu.sync_copy(etT.at[wid, seg], et_v)
        pltpu.sync_copy(dstT.at[wid, seg], dst_v)

        @pl.loop(0, SCH)
        def _(g):
            for j in range(K // 16):
                sl = pl.ds(j * 16, 16)
                idx_v[g, sl] = et_v[g, sl] * N + src_v[g, sl]

        pltpu.async_copy(table.at[idx_v.at[0]], gA, semA)

        @pl.loop(0, SCH)
        def _(g):
            nxt = g + 1

            @pl.when(jnp.logical_and(nxt < SCH, nxt % 2 == 0))
            def _():
                pltpu.async_copy(table.at[idx_v.at[nxt]], gA, semA)

            @pl.when(jnp.logical_and(nxt < SCH, nxt % 2 == 1))
            def _():
                pltpu.async_copy(table.at[idx_v.at[nxt]], gB, semB)

            @pl.when(g % 2 == 0)
            def _():
                pltpu.make_async_copy(table.at[idx_v.at[g]], gA, semA).wait()
                pltpu.sync_copy(gA, acc.at[dst_v.at[g]], add=True)

            @pl.when(g % 2 == 1)
            def _():
                pltpu.make_async_copy(table.at[idx_v.at[g]], gB, semB).wait()
                pltpu.sync_copy(gB, acc.at[dst_v.at[g]], add=True)

    plsc.subcore_barrier()

    @pl.loop(0, RPT // ZR)
    def _(i):
        r0 = row0 + i * ZR
        pltpu.sync_copy(acc.at[pl.ds(r0, ZR)], zbuf)
        pltpu.sync_copy(zbuf, out.at[c, pl.ds(r0, ZR)])


def _sc_scatter(table, srcT, etT, dstT):
    mesh = plsc.VectorSubcoreMesh(core_axis_name="c", subcore_axis_name="s")
    kern = pl.kernel(
        _sc_body,
        out_type=jax.ShapeDtypeStruct((NC, NP, F), jnp.float32),
        mesh=mesh,
        scratch_types=[
            pltpu.VMEM((SCH, K), jnp.int32),
            pltpu.VMEM((SCH, K), jnp.int32),
            pltpu.VMEM((SCH, K), jnp.int32),
            pltpu.VMEM((SCH, K), jnp.int32),
            pltpu.VMEM((K, F), jnp.float32),
            pltpu.VMEM((K, F), jnp.float32),
            pltpu.VMEM((ZR, F), jnp.float32),
            pltpu.VMEM_SHARED((NP, F), jnp.float32),
            pltpu.SemaphoreType.DMA,
            pltpu.SemaphoreType.DMA,
        ],
    )
    return kern(table, srcT, etT, dstT)


def _combine_body(p_ref, n_ref, o_ref):
    o_ref[...] = (p_ref[0] + p_ref[1]) * n_ref[...]


def _combine(partial, norm):
    grid = (N // BN,)
    return pl.pallas_call(
        _combine_body,
        grid=grid,
        in_specs=[
            pl.BlockSpec((NC, BN, F), lambda n: (0, n, 0)),
            pl.BlockSpec((BN, 1), lambda n: (n, 0)),
        ],
        out_specs=pl.BlockSpec((BN, F), lambda n: (n, 0)),
        out_shape=jax.ShapeDtypeStruct((N, F), jnp.float32),
    )(partial, norm)


def kernel(x, edge_index, edge_type, norm, weight):
    table = _make_table(x, weight)
    srcT = edge_index[0].reshape(NW, NSEG, SCH, K)
    dstT = edge_index[1].reshape(NW, NSEG, SCH, K)
    etT = edge_type.reshape(NW, NSEG, SCH, K)
    partial = _sc_scatter(table, srcT, etT, dstT)
    return _combine(partial, norm)


# speedup vs baseline: 119.9373x; 1.0465x over previous
import functools

import jax
import jax.numpy as jnp
from jax import lax
from jax.experimental import pallas as pl
from jax.experimental.pallas import tpu as pltpu
from jax.experimental.pallas import tpu_sc as plsc

N = 10000
E = 320000
F = 128
R = 8
SUB = 16

NC = 2
NS = 16
NW = NC * NS
EPW = E // NW
K = 80
CH = EPW // K
NSEG = 5
SCH = CH // NSEG
NP = 10240
RPT = NP // NS
ZR = 16
BN = 2000


def _table_body(x_ref, w_ref, out_ref):
    r = pl.program_id(0)
    w2 = w_ref[r]
    bi = lax.broadcasted_iota(jnp.int32, (F, SUB), 0) // SUB
    cols = [jnp.where(bi == b, w2, 0.0) for b in range(F // SUB)]
    bd = jnp.concatenate(cols, axis=1)
    out_ref[...] = jnp.dot(x_ref[...], bd, preferred_element_type=jnp.float32)


def _make_table(x, weight):
    grid = (R, N // BN)
    return pl.pallas_call(
        _table_body,
        grid=grid,
        in_specs=[
            pl.BlockSpec((BN, F), lambda r, n: (n, 0)),
            pl.BlockSpec((R, F, SUB), lambda r, n: (0, 0, 0)),
        ],
        out_specs=pl.BlockSpec((BN, F), lambda r, n: (r * (N // BN) + n, 0)),
        out_shape=jax.ShapeDtypeStruct((R * N, F), jnp.float32),
    )(x, weight.reshape(R, F, SUB))


def _sc_body(table, srcT, etT, dstT, out,
             src_v, idx_v, dst_v, g0, g1, g2, zbuf, acc,
             gs0, gs1, gs2, ss0, ss1, ss2):
    gbufs = (g0, g1, g2)
    gsems = (gs0, gs1, gs2)
    ssems = (ss0, ss1, ss2)
    c = lax.axis_index("c")
    s = lax.axis_index("s")
    wid = c * NS + s
    row0 = s * RPT

    zeros16 = jnp.zeros((16,), jnp.float32)

    @pl.loop(0, ZR)
    def _(r):
        for j in range(F // 16):
            zbuf[r, pl.ds(j * 16, 16)] = zeros16

    @pl.loop(0, RPT // ZR)
    def _(i):
        pltpu.sync_copy(zbuf, acc.at[pl.ds(row0 + i * ZR, ZR)])

    plsc.subcore_barrier()

    @pl.loop(0, NSEG)
    def _(seg):
        pltpu.sync_copy(srcT.at[wid, seg], src_v)
        pltpu.sync_copy(etT.at[wid, seg], idx_v)
        pltpu.sync_copy(dstT.at[wid, seg], dst_v)

        @pl.loop(0, SCH)
        def _(g):
            for j in range(K // 16):
                sl = pl.ds(j * 16, 16)
                idx_v[g, sl] = idx_v[g, sl] * N + src_v[g, sl]

        pltpu.async_copy(table.at[idx_v.at[0]], g0, gs0)

        @pl.loop(0, SCH)
        def _(g):
            for b in range(3):
                @pl.when(g % 3 == b)
                def _(b=b):
                    nb = (b + 1) % 3

                    @pl.when(g + 1 < SCH)
                    def _():
                        @pl.when(g >= 2)
                        def _():
                            pltpu.make_async_copy(
                                gbufs[nb], acc.at[dst_v.at[g - 2]],
                                ssems[nb]).wait()
                        pltpu.async_copy(
                            table.at[idx_v.at[g + 1]], gbufs[nb], gsems[nb])

                    pltpu.make_async_copy(
                        table.at[idx_v.at[g]], gbufs[b], gsems[b]).wait()
                    pltpu.async_copy(
                        gbufs[b], acc.at[dst_v.at[g]], ssems[b], add=True)

        pltpu.make_async_copy(
            gbufs[(SCH - 2) % 3], acc.at[dst_v.at[SCH - 2]],
            ssems[(SCH - 2) % 3]).wait()
        pltpu.make_async_copy(
            gbufs[(SCH - 1) % 3], acc.at[dst_v.at[SCH - 1]],
            ssems[(SCH - 1) % 3]).wait()

    plsc.subcore_barrier()

    @pl.loop(0, RPT // ZR)
    def _(i):
        r0 = row0 + i * ZR
        pltpu.sync_copy(acc.at[pl.ds(r0, ZR)], zbuf)
        pltpu.sync_copy(zbuf, out.at[c, pl.ds(r0, ZR)])


def _sc_scatter(table, srcT, etT, dstT):
    mesh = plsc.VectorSubcoreMesh(core_axis_name="c", subcore_axis_name="s")
    kern = pl.kernel(
        _sc_body,
        out_type=jax.ShapeDtypeStruct((NC, NP, F), jnp.float32),
        mesh=mesh,
        scratch_types=[
            pltpu.VMEM((SCH, K), jnp.int32),
            pltpu.VMEM((SCH, K), jnp.int32),
            pltpu.VMEM((SCH, K), jnp.int32),
            pltpu.VMEM((K, F), jnp.float32),
            pltpu.VMEM((K, F), jnp.float32),
            pltpu.VMEM((K, F), jnp.float32),
            pltpu.VMEM((ZR, F), jnp.float32),
            pltpu.VMEM_SHARED((NP, F), jnp.float32),
            pltpu.SemaphoreType.DMA,
            pltpu.SemaphoreType.DMA,
            pltpu.SemaphoreType.DMA,
            pltpu.SemaphoreType.DMA,
            pltpu.SemaphoreType.DMA,
            pltpu.SemaphoreType.DMA,
        ],
    )
    return kern(table, srcT, etT, dstT)


def _combine_body(p_ref, n_ref, o_ref):
    o_ref[...] = (p_ref[0] + p_ref[1]) * n_ref[...]


def _combine(partial, norm):
    grid = (N // BN,)
    return pl.pallas_call(
        _combine_body,
        grid=grid,
        in_specs=[
            pl.BlockSpec((NC, BN, F), lambda n: (0, n, 0)),
            pl.BlockSpec((BN, 1), lambda n: (n, 0)),
        ],
        out_specs=pl.BlockSpec((BN, F), lambda n: (n, 0)),
        out_shape=jax.ShapeDtypeStruct((N, F), jnp.float32),
    )(partial, norm)


def kernel(x, edge_index, edge_type, norm, weight):
    table = _make_table(x, weight)
    srcT = edge_index[0].reshape(NW, NSEG, SCH, K)
    dstT = edge_index[1].reshape(NW, NSEG, SCH, K)
    etT = edge_type.reshape(NW, NSEG, SCH, K)
    partial = _sc_scatter(table, srcT, etT, dstT)
    return _combine(partial, norm)


# speedup vs baseline: 139.6553x; 1.1644x over previous
import functools

import jax
import jax.numpy as jnp
from jax import lax
from jax.experimental import pallas as pl
from jax.experimental.pallas import tpu as pltpu
from jax.experimental.pallas import tpu_sc as plsc

N = 10000
E = 320000
F = 128
R = 8
SUB = 16

NC = 2
NS = 16
NW = NC * NS
EPW = E // NW
K = 80
CH = EPW // K
NSEG = 5
SCH = CH // NSEG
NP = 10240
RPT = NP // NS
ZR = 16
BN = 2000


def _table_body(x_ref, w_ref, out_ref):
    bi = lax.broadcasted_iota(jnp.int32, (F, SUB), 0) // SUB
    cols = []
    for r in range(R):
        w2 = w_ref[r]
        cols.extend(jnp.where(bi == b, w2, 0.0) for b in range(F // SUB))
    bd = jnp.concatenate(cols, axis=1)
    y = jnp.dot(x_ref[...], bd, preferred_element_type=jnp.float32)
    for r in range(R):
        out_ref[r] = y[:, r * F:(r + 1) * F]


def _make_table(x, weight):
    grid = (N // BN,)
    return pl.pallas_call(
        _table_body,
        grid=grid,
        in_specs=[
            pl.BlockSpec((BN, F), lambda n: (n, 0)),
            pl.BlockSpec((R, F, SUB), lambda n: (0, 0, 0)),
        ],
        out_specs=pl.BlockSpec((R, BN, F), lambda n: (0, n, 0)),
        out_shape=jax.ShapeDtypeStruct((R, N, F), jnp.float32),
    )(x, weight.reshape(R, F, SUB))


def _sc_body(table, srcT, etT, dstT, out,
             src_v, idx_v, dst_v, g0, g1, g2, zbuf, acc,
             gs0, gs1, gs2, ss0, ss1, ss2):
    gbufs = (g0, g1, g2)
    gsems = (gs0, gs1, gs2)
    ssems = (ss0, ss1, ss2)
    c = lax.axis_index("c")
    s = lax.axis_index("s")
    wid = c * NS + s
    row0 = s * RPT

    zeros16 = jnp.zeros((16,), jnp.float32)

    @pl.loop(0, ZR)
    def _(r):
        for j in range(F // 16):
            zbuf[r, pl.ds(j * 16, 16)] = zeros16

    @pl.loop(0, RPT // ZR)
    def _(i):
        pltpu.sync_copy(zbuf, acc.at[pl.ds(row0 + i * ZR, ZR)])

    plsc.subcore_barrier()

    @pl.loop(0, NSEG)
    def _(seg):
        pltpu.sync_copy(srcT.at[wid, seg], src_v)
        pltpu.sync_copy(etT.at[wid, seg], idx_v)
        pltpu.sync_copy(dstT.at[wid, seg], dst_v)

        @pl.loop(0, SCH)
        def _(g):
            for j in range(K // 16):
                sl = pl.ds(j * 16, 16)
                idx_v[g, sl] = idx_v[g, sl] * N + src_v[g, sl]

        pltpu.async_copy(table.at[idx_v.at[0]], g0, gs0)

        @pl.loop(0, SCH)
        def _(g):
            for b in range(3):
                @pl.when(g % 3 == b)
                def _(b=b):
                    nb = (b + 1) % 3

                    @pl.when(g + 1 < SCH)
                    def _():
                        @pl.when(g >= 2)
                        def _():
                            pltpu.make_async_copy(
                                gbufs[nb], acc.at[dst_v.at[g - 2]],
                                ssems[nb]).wait()
                        pltpu.async_copy(
                            table.at[idx_v.at[g + 1]], gbufs[nb], gsems[nb])

                    pltpu.make_async_copy(
                        table.at[idx_v.at[g]], gbufs[b], gsems[b]).wait()
                    pltpu.async_copy(
                        gbufs[b], acc.at[dst_v.at[g]], ssems[b], add=True)

        pltpu.make_async_copy(
            gbufs[(SCH - 2) % 3], acc.at[dst_v.at[SCH - 2]],
            ssems[(SCH - 2) % 3]).wait()
        pltpu.make_async_copy(
            gbufs[(SCH - 1) % 3], acc.at[dst_v.at[SCH - 1]],
            ssems[(SCH - 1) % 3]).wait()

    plsc.subcore_barrier()

    @pl.loop(0, RPT // ZR)
    def _(i):
        r0 = row0 + i * ZR
        pltpu.sync_copy(acc.at[pl.ds(r0, ZR)], zbuf)
        pltpu.sync_copy(zbuf, out.at[c, pl.ds(r0, ZR)])


def _sc_scatter(table, srcT, etT, dstT):
    mesh = plsc.VectorSubcoreMesh(core_axis_name="c", subcore_axis_name="s")
    kern = pl.kernel(
        _sc_body,
        out_type=jax.ShapeDtypeStruct((NC, NP, F), jnp.float32),
        mesh=mesh,
        scratch_types=[
            pltpu.VMEM((SCH, K), jnp.int32),
            pltpu.VMEM((SCH, K), jnp.int32),
            pltpu.VMEM((SCH, K), jnp.int32),
            pltpu.VMEM((K, F), jnp.float32),
            pltpu.VMEM((K, F), jnp.float32),
            pltpu.VMEM((K, F), jnp.float32),
            pltpu.VMEM((ZR, F), jnp.float32),
            pltpu.VMEM_SHARED((NP, F), jnp.float32),
            pltpu.SemaphoreType.DMA,
            pltpu.SemaphoreType.DMA,
            pltpu.SemaphoreType.DMA,
            pltpu.SemaphoreType.DMA,
            pltpu.SemaphoreType.DMA,
            pltpu.SemaphoreType.DMA,
        ],
    )
    return kern(table, srcT, etT, dstT)


def _combine_body(p_ref, n_ref, o_ref):
    o_ref[...] = (p_ref[0] + p_ref[1]) * n_ref[...]


def _combine(partial, norm):
    grid = (N // BN,)
    return pl.pallas_call(
        _combine_body,
        grid=grid,
        in_specs=[
            pl.BlockSpec((NC, BN, F), lambda n: (0, n, 0)),
            pl.BlockSpec((BN, 1), lambda n: (n, 0)),
        ],
        out_specs=pl.BlockSpec((BN, F), lambda n: (n, 0)),
        out_shape=jax.ShapeDtypeStruct((N, F), jnp.float32),
    )(partial, norm)


def kernel(x, edge_index, edge_type, norm, weight):
    table = _make_table(x, weight).reshape(R * N, F)
    srcT = edge_index[0].reshape(NW, NSEG, SCH, K)
    dstT = edge_index[1].reshape(NW, NSEG, SCH, K)
    etT = edge_type.reshape(NW, NSEG, SCH, K)
    partial = _sc_scatter(table, srcT, etT, dstT)
    return _combine(partial, norm)


# speedup vs baseline: 145.7946x; 1.0440x over previous
import functools

import jax
import jax.numpy as jnp
from jax import lax
from jax.experimental import pallas as pl
from jax.experimental.pallas import tpu as pltpu
from jax.experimental.pallas import tpu_sc as plsc

N = 10000
E = 320000
F = 128
R = 8
SUB = 16

NC = 2
NS = 16
NW = NC * NS
EPW = E // NW
K = 80
CH = EPW // K
NSEG = 5
SCH = CH // NSEG
NP = 10240
RPT = NP // NS
ZR = 16
BN = 2000


def _table_body(x_ref, w_ref, out_ref):
    bi = lax.broadcasted_iota(jnp.int32, (F, SUB), 0) // SUB
    cols = []
    for r in range(R):
        w2 = w_ref[r]
        cols.extend(jnp.where(bi == b, w2, 0.0) for b in range(F // SUB))
    bd = jnp.concatenate(cols, axis=1)
    y = jnp.dot(x_ref[...], bd, preferred_element_type=jnp.float32)
    for r in range(R):
        out_ref[r] = y[:, r * F:(r + 1) * F]


def _make_table(x, weight):
    grid = (N // BN,)
    return pl.pallas_call(
        _table_body,
        grid=grid,
        in_specs=[
            pl.BlockSpec((BN, F), lambda n: (n, 0)),
            pl.BlockSpec((R, F, SUB), lambda n: (0, 0, 0)),
        ],
        out_specs=pl.BlockSpec((R, BN, F), lambda n: (0, n, 0)),
        out_shape=jax.ShapeDtypeStruct((R, N, F), jnp.float32),
    )(x, weight.reshape(R, F, SUB))


def _sc_body(table, srcT, etT, dstT, out,
             src_v, idx_v, dst_v, g0, g1, g2, zbuf, acc,
             gs0, gs1, gs2, ss0, ss1, ss2):
    gbufs = (g0, g1, g2)
    gsems = (gs0, gs1, gs2)
    ssems = (ss0, ss1, ss2)
    c = lax.axis_index("c")
    s = lax.axis_index("s")
    wid = c * NS + s
    row0 = s * RPT

    zeros16 = jnp.zeros((16,), jnp.float32)

    @pl.loop(0, ZR)
    def _(r):
        for j in range(F // 16):
            zbuf[r, pl.ds(j * 16, 16)] = zeros16

    @pl.loop(0, RPT // ZR)
    def _(i):
        pltpu.async_copy(zbuf, acc.at[pl.ds(row0 + i * ZR, ZR)], gs0)

    @pl.loop(0, RPT // ZR)
    def _(i):
        pltpu.make_async_copy(zbuf, acc.at[pl.ds(row0 + i * ZR, ZR)], gs0).wait()

    plsc.subcore_barrier()

    @pl.loop(0, NSEG)
    def _(seg):
        pltpu.sync_copy(srcT.at[wid, seg], src_v)
        pltpu.sync_copy(etT.at[wid, seg], idx_v)
        pltpu.sync_copy(dstT.at[wid, seg], dst_v)

        @pl.loop(0, SCH)
        def _(g):
            for j in range(K // 16):
                sl = pl.ds(j * 16, 16)
                idx_v[g, sl] = idx_v[g, sl] * N + src_v[g, sl]

        pltpu.async_copy(table.at[idx_v.at[0]], g0, gs0)

        @pl.loop(0, SCH)
        def _(g):
            for b in range(3):
                @pl.when(g % 3 == b)
                def _(b=b):
                    nb = (b + 1) % 3

                    @pl.when(g + 1 < SCH)
                    def _():
                        @pl.when(g >= 2)
                        def _():
                            pltpu.make_async_copy(
                                gbufs[nb], acc.at[dst_v.at[g - 2]],
                                ssems[nb]).wait()
                        pltpu.async_copy(
                            table.at[idx_v.at[g + 1]], gbufs[nb], gsems[nb])

                    pltpu.make_async_copy(
                        table.at[idx_v.at[g]], gbufs[b], gsems[b]).wait()
                    pltpu.async_copy(
                        gbufs[b], acc.at[dst_v.at[g]], ssems[b], add=True)

        pltpu.make_async_copy(
            gbufs[(SCH - 2) % 3], acc.at[dst_v.at[SCH - 2]],
            ssems[(SCH - 2) % 3]).wait()
        pltpu.make_async_copy(
            gbufs[(SCH - 1) % 3], acc.at[dst_v.at[SCH - 1]],
            ssems[(SCH - 1) % 3]).wait()

    plsc.subcore_barrier()

    pltpu.sync_copy(acc.at[pl.ds(row0, RPT)], out.at[c, pl.ds(row0, RPT)])


def _sc_scatter(table, srcT, etT, dstT):
    mesh = plsc.VectorSubcoreMesh(core_axis_name="c", subcore_axis_name="s")
    kern = pl.kernel(
        _sc_body,
        out_type=jax.ShapeDtypeStruct((NC, NP, F), jnp.float32),
        mesh=mesh,
        scratch_types=[
            pltpu.VMEM((SCH, K), jnp.int32),
            pltpu.VMEM((SCH, K), jnp.int32),
            pltpu.VMEM((SCH, K), jnp.int32),
            pltpu.VMEM((K, F), jnp.float32),
            pltpu.VMEM((K, F), jnp.float32),
            pltpu.VMEM((K, F), jnp.float32),
            pltpu.VMEM((ZR, F), jnp.float32),
            pltpu.VMEM_SHARED((NP, F), jnp.float32),
            pltpu.SemaphoreType.DMA,
            pltpu.SemaphoreType.DMA,
            pltpu.SemaphoreType.DMA,
            pltpu.SemaphoreType.DMA,
            pltpu.SemaphoreType.DMA,
            pltpu.SemaphoreType.DMA,
        ],
    )
    return kern(table, srcT, etT, dstT)


def _combine_body(p_ref, n_ref, o_ref):
    o_ref[...] = (p_ref[0] + p_ref[1]) * n_ref[...]


def _combine(partial, norm):
    grid = (N // BN,)
    return pl.pallas_call(
        _combine_body,
        grid=grid,
        in_specs=[
            pl.BlockSpec((NC, BN, F), lambda n: (0, n, 0)),
            pl.BlockSpec((BN, 1), lambda n: (n, 0)),
        ],
        out_specs=pl.BlockSpec((BN, F), lambda n: (n, 0)),
        out_shape=jax.ShapeDtypeStruct((N, F), jnp.float32),
    )(partial, norm)


def kernel(x, edge_index, edge_type, norm, weight):
    table = _make_table(x, weight).reshape(R * N, F)
    srcT = edge_index[0].reshape(NW, NSEG, SCH, K)
    dstT = edge_index[1].reshape(NW, NSEG, SCH, K)
    etT = edge_type.reshape(NW, NSEG, SCH, K)
    partial = _sc_scatter(table, srcT, etT, dstT)
    return _combine(partial, norm)
